# baseline (speedup 1.0000x reference)
import functools

import jax
import jax.numpy as jnp
from jax import lax
from jax.experimental import pallas as pl
from jax.experimental.pallas import tpu as pltpu
from jax.experimental.pallas import tpu_sc as plsc

B, T, D = 2, 2048, 1024
E, K = 8, 2
H = D * 4
N = B * T
A = N * K
TM = 256
PT = A // TM + E
S = PT * TM
GT = 512
NSC = 16
NCORE = 2
NW = NSC * NCORE

_f32 = jnp.float32
_i32 = jnp.int32


def _gate_body(xf_ref, gw_ref, gb_ref,
               idx0_ref, idx1_ref, p0_ref, p1_ref, te_ref, act_ref, aux_ref,
               hist_ref, cnt_acc):
    t = pl.program_id(0)
    logits = jnp.dot(xf_ref[...].astype(jnp.bfloat16),
                     gw_ref[...].astype(jnp.bfloat16),
                     preferred_element_type=_f32) + gb_ref[...]
    iota_e = lax.broadcasted_iota(_i32, (GT, E), 1)
    m1 = jnp.max(logits, axis=1, keepdims=True)
    i1 = jnp.min(jnp.where(logits == m1, iota_e, E), axis=1, keepdims=True)
    lm = jnp.where(iota_e == i1, -jnp.inf, logits)
    m2 = jnp.max(lm, axis=1, keepdims=True)
    i2 = jnp.min(jnp.where(lm == m2, iota_e, E), axis=1, keepdims=True)
    b = jnp.exp(m2 - m1)
    p0 = 1.0 / (1.0 + b)
    p1 = b / (1.0 + b)
    idx0_ref[...] = i1
    idx1_ref[...] = i2
    p0_ref[...] = p0
    p1_ref[...] = p1

    onehot = (iota_e == i1).astype(_f32) + (iota_e == i2).astype(_f32)
    csum = jnp.sum(onehot, axis=0, keepdims=True)
    cnt_acc[...] = jnp.where(t == 0, csum, cnt_acc[...] + csum)

    z = jnp.zeros((1, 16 - E), _f32)
    h0 = jnp.sum(onehot[:GT // 2], axis=0, keepdims=True)
    h1 = jnp.sum(onehot[GT // 2:], axis=0, keepdims=True)
    hist_ref[0, 0:1, :] = jnp.concatenate([h0, z], axis=1).astype(_i32)
    hist_ref[0, 1:2, :] = jnp.concatenate([h1, z], axis=1).astype(_i32)

    @pl.when(t == pl.num_programs(0) - 1)
    def _():
        c = cnt_acc[...]
        aux_ref[...] = jnp.sum((c / float(A) - 1.0 / E) ** 2,
                               axis=1, keepdims=True)
        ci = c.astype(_i32)
        ptl = (ci + (TM - 1)) // TM
        ptb = jnp.broadcast_to(ptl, (E, E))
        ii = lax.broadcasted_iota(_i32, (E, E), 0)
        jj = lax.broadcasted_iota(_i32, (E, E), 1)
        ends_col = jnp.sum(jnp.where(jj <= ii, ptb, 0), axis=1, keepdims=True)
        pt_col = jnp.sum(jnp.where(jj == ii, ptb, 0), axis=1, keepdims=True)
        iota_col = lax.broadcasted_iota(_i32, (E, 1), 0)
        total_tiles = jnp.max(ends_col, axis=0, keepdims=True)
        last_e = jnp.max(jnp.where(pt_col > 0, iota_col, 0),
                         axis=0, keepdims=True)
        tt = lax.broadcasted_iota(_i32, (E, 128), 1)
        te_row = jnp.sum((tt >= ends_col).astype(_i32), axis=0, keepdims=True)
        t1 = lax.broadcasted_iota(_i32, (1, 128), 1)
        act = t1 < total_tiles
        te_ref[...] = jnp.where(act, jnp.minimum(te_row, E - 1), last_e)
        act_ref[...] = act.astype(_i32)


def _gate(xf, gate_w, gate_b):
    grid = (N // GT,)
    return pl.pallas_call(
        _gate_body,
        grid=grid,
        in_specs=[
            pl.BlockSpec((GT, D), lambda t: (t, 0)),
            pl.BlockSpec((D, E), lambda t: (0, 0)),
            pl.BlockSpec((1, E), lambda t: (0, 0)),
        ],
        out_specs=[
            pl.BlockSpec((GT, 1), lambda t: (t, 0)),
            pl.BlockSpec((GT, 1), lambda t: (t, 0)),
            pl.BlockSpec((GT, 1), lambda t: (t, 0)),
            pl.BlockSpec((GT, 1), lambda t: (t, 0)),
            pl.BlockSpec((1, 128), lambda t: (0, 0)),
            pl.BlockSpec((1, 128), lambda t: (0, 0)),
            pl.BlockSpec((1, 1), lambda t: (0, 0)),
            pl.BlockSpec((1, 2, 16), lambda t: (t, 0, 0)),
        ],
        out_shape=[
            jax.ShapeDtypeStruct((N, 1), _i32),
            jax.ShapeDtypeStruct((N, 1), _i32),
            jax.ShapeDtypeStruct((N, 1), _f32),
            jax.ShapeDtypeStruct((N, 1), _f32),
            jax.ShapeDtypeStruct((1, 128), _i32),
            jax.ShapeDtypeStruct((1, 128), _i32),
            jax.ShapeDtypeStruct((1, 1), _f32),
            jax.ShapeDtypeStruct((N // GT, 2, 16), _i32),
        ],
        scratch_shapes=[pltpu.VMEM((1, E), _f32)],
    )(xf, gate_w, gate_b.reshape(1, E))


def _routing_body(idx0_h, idx1_h, p0_h, p1_h, hist_h,
                  pos0_h, pos1_h, tos_h, psc_h,
                  e0v, e1v, pflat, posflat, posb2d, tokb, pb2d,
                  allh_v, sem):
    cid = lax.axis_index("c")
    sid = lax.axis_index("s")
    lane = lax.broadcasted_iota(_i32, (16,), 0)

    @pl.when(cid == 0)
    def _():
        base = sid * 256
        pltpu.sync_copy(idx0_h.at[pl.ds(base, 256)], e0v)
        pltpu.sync_copy(idx1_h.at[pl.ds(base, 256)], e1v)
        pltpu.sync_copy(hist_h, allh_v)

        total = jnp.zeros((16,), _i32)
        mybase = jnp.zeros((16,), _i32)
        for sp in range(NSC):
            row = allh_v[pl.ds(sp * 16, 16)]
            total = total + row
            before = jnp.full((16,), sp, _i32) < jnp.full((16,), sid, _i32)
            mybase = mybase + jnp.where(before, row, 0)
        ptl = (total + (TM - 1)) // TM
        pstart = (plsc.cumsum(ptl) - ptl) * TM
        ebase = pstart + mybase

        def scan(ev, run0):
            def chunk(c, runs):
                v = ev[pl.ds(c * 16, 16)]
                pos = jnp.zeros((16,), _i32)
                new_runs = []
                for e in range(E):
                    m = v == e
                    mi = m.astype(_i32)
                    cs = lax.cumsum(mi)
                    pos = jnp.where(m, runs[e] + cs - 1, pos)
                    new_runs.append(runs[e] + jnp.sum(mi))
                posflat[pl.ds(c * 16, 16)] = pos
                return tuple(new_runs)
            return lax.fori_loop(0, 16, chunk, run0)

        run0 = tuple(ebase[e] for e in range(E))

        for j in range(2):
            for cc in range(8):
                tokb[j, pl.ds(cc * 16, 16)] = (base + j * 128 + cc * 16) + lane

        def emit(pos_out_h, p_h):
            for j in range(2):
                for cc in range(8):
                    posb2d[j, pl.ds(cc * 16, 16)] = posflat[
                        pl.ds(j * 128 + cc * 16, 16)]
            pltpu.sync_copy(posb2d, pos_out_h.at[pl.ds(sid * 2, 2)])
            pltpu.sync_copy(p_h.at[pl.ds(base, 256)], pflat)
            for j in range(2):
                for cc in range(8):
                    pb2d[j, pl.ds(cc * 16, 16)] = pflat[
                        pl.ds(j * 128 + cc * 16, 16)]
            for j in range(2):
                pltpu.async_copy(tokb.at[j], tos_h.at[posb2d.at[j]],
                                 sem).wait()
                pltpu.async_copy(pb2d.at[j], psc_h.at[posb2d.at[j]],
                                 sem).wait()

        run1 = scan(e0v, run0)
        emit(pos0_h, p0_h)
        scan(e1v, run1)
        emit(pos1_h, p1_h)


def _routing(idx0, idx1, p0, p1, hist):
    mesh = plsc.VectorSubcoreMesh(core_axis_name="c", subcore_axis_name="s")
    f = pl.kernel(
        _routing_body,
        out_type=[
            jax.ShapeDtypeStruct((N // 128, 128), _i32),
            jax.ShapeDtypeStruct((N // 128, 128), _i32),
            jax.ShapeDtypeStruct((S,), _i32),
            jax.ShapeDtypeStruct((S,), _f32),
        ],
        mesh=mesh,
        scratch_types=[
            pltpu.VMEM((256,), _i32),
            pltpu.VMEM((256,), _i32),
            pltpu.VMEM((256,), _f32),
            pltpu.VMEM((256,), _i32),
            pltpu.VMEM((2, 128), _i32),
            pltpu.VMEM((2, 128), _i32),
            pltpu.VMEM((2, 128), _f32),
            pltpu.VMEM((NSC * 16,), _i32),
            pltpu.SemaphoreType.DMA,
        ],
        compiler_params=pltpu.CompilerParams(needs_layout_passes=False),
    )
    return f(idx0, idx1, p0, p1, hist)


_GR = 10
_GCH = (S // NW) // _GR


def _gather_body(xf_h, tos_h, xs_h, tosv, clv2, rows0, rows1,
                 sg0, sg1, sc0, sc1):
    cid = lax.axis_index("c")
    sid = lax.axis_index("s")
    wid = sid * NCORE + cid
    per = S // NW
    base = wid * per
    pltpu.sync_copy(tos_h.at[pl.ds(base, per)], tosv)

    for c in range(per // 16):
        v = tosv[pl.ds(c * 16, 16)]
        clv2[c // 2, pl.ds((c % 2) * 16, 16)] = jnp.minimum(
            jnp.maximum(v, 0), N - 1)

    rows = (rows0, rows1)
    sg = (sg0, sg1)
    sc = (sc0, sc1)
    ga = {}
    cp = {}

    def copy_out(r):
        b = r & 1
        ga[r].wait()
        cp[r] = pltpu.async_copy(
            rows[b], xs_h.at[pl.ds(base + r * _GCH, _GCH), :], sc[b])

    for r in range(_GR):
        b = r & 1
        if r >= 2:
            cp[r - 2].wait()
        ga[r] = pltpu.async_copy(xf_h.at[clv2.at[r]], rows[b], sg[b])
        if r >= 1:
            copy_out(r - 1)
    copy_out(_GR - 1)
    cp[_GR - 2].wait()
    cp[_GR - 1].wait()


def _gather(xf, tos):
    mesh = plsc.VectorSubcoreMesh(core_axis_name="c", subcore_axis_name="s")
    f = pl.kernel(
        _gather_body,
        out_type=[jax.ShapeDtypeStruct((S, D), _f32)],
        mesh=mesh,
        scratch_types=[
            pltpu.VMEM((S // NW,), _i32),
            pltpu.VMEM((_GR, _GCH), _i32),
            pltpu.VMEM((_GCH, D), _f32),
            pltpu.VMEM((_GCH, D), _f32),
            pltpu.SemaphoreType.DMA,
            pltpu.SemaphoreType.DMA,
            pltpu.SemaphoreType.DMA,
            pltpu.SemaphoreType.DMA,
        ],
        compiler_params=pltpu.CompilerParams(needs_layout_passes=False),
    )
    return f(xf, tos)[0]


NH = 4
HB = H // NH


def _ffn_body(te_ref, act_ref, xs_ref, psc_ref,
              w1_ref, w2_ref, wp_ref, b1_ref, b2_ref, bp_ref, ysin_ref,
              ys_ref):
    h = pl.program_id(0)
    t = pl.program_id(1)

    @pl.when(act_ref[t] == 1)
    def _():
        xb = xs_ref[...].astype(jnp.bfloat16)
        h1 = jnp.dot(xb, w1_ref[0],
                     preferred_element_type=_f32) + b1_ref[0]
        h2 = jnp.dot(xb, w2_ref[0],
                     preferred_element_type=_f32) + b2_ref[0]
        g = (h1 * jax.nn.sigmoid(h1)) * h2
        part = jnp.dot(g.astype(jnp.bfloat16), wp_ref[0],
                       preferred_element_type=_f32)
        acc = ysin_ref[...] + part
        final = (acc + bp_ref[0]) * psc_ref[...]
        ys_ref[...] = jnp.where(h == NH - 1, final, acc)


def _ffn(te, act, xs, psc, w1b, w2b, wpb, b1, b2, bp, ysin):
    grid_spec = pltpu.PrefetchScalarGridSpec(
        num_scalar_prefetch=2,
        grid=(NH, PT),
        in_specs=[
            pl.BlockSpec((TM, D), lambda h, t, te, a: (t, 0)),
            pl.BlockSpec((TM, 1), lambda h, t, te, a: (t, 0)),
            pl.BlockSpec((1, D, HB), lambda h, t, te, a: (te[t], 0, h)),
            pl.BlockSpec((1, D, HB), lambda h, t, te, a: (te[t], 0, h)),
            pl.BlockSpec((1, HB, D), lambda h, t, te, a: (te[t], h, 0)),
            pl.BlockSpec((1, 1, HB), lambda h, t, te, a: (te[t], 0, h)),
            pl.BlockSpec((1, 1, HB), lambda h, t, te, a: (te[t], 0, h)),
            pl.BlockSpec((1, 1, D), lambda h, t, te, a: (te[t], 0, 0)),
            pl.BlockSpec((TM, D), lambda h, t, te, a: (t, 0)),
        ],
        out_specs=pl.BlockSpec((TM, D), lambda h, t, te, a: (t, 0)),
    )
    return pl.pallas_call(
        _ffn_body,
        grid_spec=grid_spec,
        out_shape=jax.ShapeDtypeStruct((S, D), _f32),
        input_output_aliases={10: 0},
    )(te, act, xs, psc, w1b, w2b, wpb, b1, b2, bp, ysin)


_CR = 8
_CCH = (N // NW) // _CR


def _combine_body(ys_h, pos0_h, pos1_h, a0_h, a1_h, i0v, i1v,
                  a00, a01, a10, a11, sa0, sa1, sb0, sb1, so0, so1,
                  sp0, sp1):
    cid = lax.axis_index("c")
    sid = lax.axis_index("s")
    wid = sid * NCORE + cid
    pltpu.sync_copy(pos0_h.at[pl.ds(wid, 1)], i0v)
    pltpu.sync_copy(pos1_h.at[pl.ds(wid, 1)], i1v)

    a0 = (a00, a01)
    a1 = (a10, a11)
    sa = (sa0, sa1)
    sb = (sb0, sb1)
    so = (so0, so1)
    sp = (sp0, sp1)
    g0 = {}
    g1 = {}
    cp0 = {}
    cp1 = {}

    def copy_out(r):
        b = r & 1
        g0[r].wait()
        g1[r].wait()
        dst = pl.ds(wid * 128 + r * _CCH, _CCH)
        cp0[r] = pltpu.async_copy(a0[b], a0_h.at[dst, :], so[b])
        cp1[r] = pltpu.async_copy(a1[b], a1_h.at[dst, :], sp[b])

    for r in range(_CR):
        b = r & 1
        if r >= 2:
            cp0[r - 2].wait()
            cp1[r - 2].wait()
        g0[r] = pltpu.async_copy(
            ys_h.at[i0v.at[0, pl.ds(r * _CCH, _CCH)]], a0[b], sa[b])
        g1[r] = pltpu.async_copy(
            ys_h.at[i1v.at[0, pl.ds(r * _CCH, _CCH)]], a1[b], sb[b])
        if r >= 1:
            copy_out(r - 1)
    copy_out(_CR - 1)
    for r in (_CR - 2, _CR - 1):
        cp0[r].wait()
        cp1[r].wait()


def _combine(ys, pos0, pos1):
    mesh = plsc.VectorSubcoreMesh(core_axis_name="c", subcore_axis_name="s")
    f = pl.kernel(
        _combine_body,
        out_type=[
            jax.ShapeDtypeStruct((N, D), _f32),
            jax.ShapeDtypeStruct((N, D), _f32),
        ],
        mesh=mesh,
        scratch_types=[
            pltpu.VMEM((1, 128), _i32),
            pltpu.VMEM((1, 128), _i32),
            pltpu.VMEM((_CCH, D), _f32),
            pltpu.VMEM((_CCH, D), _f32),
            pltpu.VMEM((_CCH, D), _f32),
            pltpu.VMEM((_CCH, D), _f32),
            pltpu.SemaphoreType.DMA,
            pltpu.SemaphoreType.DMA,
            pltpu.SemaphoreType.DMA,
            pltpu.SemaphoreType.DMA,
            pltpu.SemaphoreType.DMA,
            pltpu.SemaphoreType.DMA,
            pltpu.SemaphoreType.DMA,
            pltpu.SemaphoreType.DMA,
        ],
        compiler_params=pltpu.CompilerParams(needs_layout_passes=False),
    )
    return f(ys, pos0, pos1)


def _add_body(a_ref, b_ref, o_ref):
    o_ref[...] = a_ref[...] + b_ref[...]


def _add(a, b):
    return pl.pallas_call(
        _add_body,
        grid=(N // GT,),
        in_specs=[pl.BlockSpec((GT, D), lambda t: (t, 0)),
                  pl.BlockSpec((GT, D), lambda t: (t, 0))],
        out_specs=pl.BlockSpec((GT, D), lambda t: (t, 0)),
        out_shape=jax.ShapeDtypeStruct((N, D), _f32),
    )(a, b)


def kernel(x, gate_w, gate_b, w1, b1, w2, b2, wp, bp):
    xf = x.reshape(N, D)
    (idx0, idx1, p0, p1, te128, act128, aux, hist) = _gate(xf, gate_w,
                                                           gate_b)
    pos0, pos1, tos, psc = _routing(idx0.reshape(N), idx1.reshape(N),
                                    p0.reshape(N), p1.reshape(N),
                                    hist.reshape(NSC * 16))
    xs = _gather(xf, tos)
    ys = _ffn(te128[0, :PT], act128[0, :PT], xs, psc.reshape(S, 1),
              w1.astype(jnp.bfloat16), w2.astype(jnp.bfloat16),
              wp.astype(jnp.bfloat16), b1.reshape(E, 1, H),
              b2.reshape(E, 1, H), bp.reshape(E, 1, D),
              jnp.zeros((S, D), _f32))
    a0, a1 = _combine(ys, pos0, pos1)
    out = _add(a0, a1)
    return out.reshape(B, T, D), aux[0, 0]


# speedup vs baseline: 1.3683x; 1.3683x over previous
import functools

import jax
import jax.numpy as jnp
from jax import lax
from jax.experimental import pallas as pl
from jax.experimental.pallas import tpu as pltpu
from jax.experimental.pallas import tpu_sc as plsc

B, T, D = 2, 2048, 1024
E, K = 8, 2
H = D * 4
N = B * T
A = N * K
TM = 256
PT = A // TM + E
S = PT * TM
GT = 512
NSC = 16
NCORE = 2
NW = NSC * NCORE

_f32 = jnp.float32
_i32 = jnp.int32


def _gate_body(xf_ref, gw_ref, gb_ref,
               idx0_ref, idx1_ref, p0_ref, p1_ref, te_ref, act_ref, aux_ref,
               hist_ref, cnt_acc):
    t = pl.program_id(0)
    logits = jnp.dot(xf_ref[...].astype(jnp.bfloat16),
                     gw_ref[...].astype(jnp.bfloat16),
                     preferred_element_type=_f32) + gb_ref[...]
    iota_e = lax.broadcasted_iota(_i32, (GT, E), 1)
    m1 = jnp.max(logits, axis=1, keepdims=True)
    i1 = jnp.min(jnp.where(logits == m1, iota_e, E), axis=1, keepdims=True)
    lm = jnp.where(iota_e == i1, -jnp.inf, logits)
    m2 = jnp.max(lm, axis=1, keepdims=True)
    i2 = jnp.min(jnp.where(lm == m2, iota_e, E), axis=1, keepdims=True)
    b = jnp.exp(m2 - m1)
    p0 = 1.0 / (1.0 + b)
    p1 = b / (1.0 + b)
    idx0_ref[...] = i1
    idx1_ref[...] = i2
    p0_ref[...] = p0
    p1_ref[...] = p1

    onehot = (iota_e == i1).astype(_f32) + (iota_e == i2).astype(_f32)
    csum = jnp.sum(onehot, axis=0, keepdims=True)
    cnt_acc[...] = jnp.where(t == 0, csum, cnt_acc[...] + csum)

    z = jnp.zeros((1, 16 - E), _f32)
    h0 = jnp.sum(onehot[:GT // 2], axis=0, keepdims=True)
    h1 = jnp.sum(onehot[GT // 2:], axis=0, keepdims=True)
    hist_ref[0, 0:1, :] = jnp.concatenate([h0, z], axis=1).astype(_i32)
    hist_ref[0, 1:2, :] = jnp.concatenate([h1, z], axis=1).astype(_i32)

    @pl.when(t == pl.num_programs(0) - 1)
    def _():
        c = cnt_acc[...]
        aux_ref[...] = jnp.sum((c / float(A) - 1.0 / E) ** 2,
                               axis=1, keepdims=True)
        ci = c.astype(_i32)
        ptl = (ci + (TM - 1)) // TM
        ptb = jnp.broadcast_to(ptl, (E, E))
        ii = lax.broadcasted_iota(_i32, (E, E), 0)
        jj = lax.broadcasted_iota(_i32, (E, E), 1)
        ends_col = jnp.sum(jnp.where(jj <= ii, ptb, 0), axis=1, keepdims=True)
        pt_col = jnp.sum(jnp.where(jj == ii, ptb, 0), axis=1, keepdims=True)
        iota_col = lax.broadcasted_iota(_i32, (E, 1), 0)
        total_tiles = jnp.max(ends_col, axis=0, keepdims=True)
        last_e = jnp.max(jnp.where(pt_col > 0, iota_col, 0),
                         axis=0, keepdims=True)
        tt = lax.broadcasted_iota(_i32, (E, 128), 1)
        te_row = jnp.sum((tt >= ends_col).astype(_i32), axis=0, keepdims=True)
        t1 = lax.broadcasted_iota(_i32, (1, 128), 1)
        act = t1 < total_tiles
        te_ref[...] = jnp.where(act, jnp.minimum(te_row, E - 1), last_e)
        act_ref[...] = act.astype(_i32)


def _gate(xf, gate_w, gate_b):
    grid = (N // GT,)
    return pl.pallas_call(
        _gate_body,
        grid=grid,
        in_specs=[
            pl.BlockSpec((GT, D), lambda t: (t, 0)),
            pl.BlockSpec((D, E), lambda t: (0, 0)),
            pl.BlockSpec((1, E), lambda t: (0, 0)),
        ],
        out_specs=[
            pl.BlockSpec((GT, 1), lambda t: (t, 0)),
            pl.BlockSpec((GT, 1), lambda t: (t, 0)),
            pl.BlockSpec((GT, 1), lambda t: (t, 0)),
            pl.BlockSpec((GT, 1), lambda t: (t, 0)),
            pl.BlockSpec((1, 128), lambda t: (0, 0)),
            pl.BlockSpec((1, 128), lambda t: (0, 0)),
            pl.BlockSpec((1, 1), lambda t: (0, 0)),
            pl.BlockSpec((1, 2, 16), lambda t: (t, 0, 0)),
        ],
        out_shape=[
            jax.ShapeDtypeStruct((N, 1), _i32),
            jax.ShapeDtypeStruct((N, 1), _i32),
            jax.ShapeDtypeStruct((N, 1), _f32),
            jax.ShapeDtypeStruct((N, 1), _f32),
            jax.ShapeDtypeStruct((1, 128), _i32),
            jax.ShapeDtypeStruct((1, 128), _i32),
            jax.ShapeDtypeStruct((1, 1), _f32),
            jax.ShapeDtypeStruct((N // GT, 2, 16), _i32),
        ],
        scratch_shapes=[pltpu.VMEM((1, E), _f32)],
    )(xf, gate_w, gate_b.reshape(1, E))


def _routing_body(idx0_h, idx1_h, p0_h, p1_h, hist_h,
                  pos0_h, pos1_h, tos_h, psc_h,
                  e0v, e1v, pflat, posflat, posb2d, tokb, pb2d,
                  allh_v, zv, sem):
    cid = lax.axis_index("c")
    sid = lax.axis_index("s")
    lane = lax.broadcasted_iota(_i32, (16,), 0)

    @pl.when(cid == 0)
    def _():
        base = sid * 256

        zper = S // NSC
        def zchunk(c, _):
            slot = sid * zper + c * 16 + lane
            zv[pl.ds(c * 16, 16)] = slot & (N - 1)
            return 0
        lax.fori_loop(0, zper // 16, zchunk, 0)
        pltpu.sync_copy(zv, tos_h.at[pl.ds(sid * zper, zper)])
        plsc.subcore_barrier()

        pltpu.sync_copy(idx0_h.at[pl.ds(base, 256)], e0v)
        pltpu.sync_copy(idx1_h.at[pl.ds(base, 256)], e1v)
        pltpu.sync_copy(hist_h, allh_v)

        total = jnp.zeros((16,), _i32)
        mybase = jnp.zeros((16,), _i32)
        for sp in range(NSC):
            row = allh_v[pl.ds(sp * 16, 16)]
            total = total + row
            before = jnp.full((16,), sp, _i32) < jnp.full((16,), sid, _i32)
            mybase = mybase + jnp.where(before, row, 0)
        ptl = (total + (TM - 1)) // TM
        pstart = (plsc.cumsum(ptl) - ptl) * TM
        ebase = pstart + mybase

        def scan(ev, run0):
            def chunk(c, runs):
                v = ev[pl.ds(c * 16, 16)]
                pos = jnp.zeros((16,), _i32)
                new_runs = []
                for e in range(E):
                    m = v == e
                    mi = m.astype(_i32)
                    cs = lax.cumsum(mi)
                    pos = jnp.where(m, runs[e] + cs - 1, pos)
                    new_runs.append(runs[e] + jnp.sum(mi))
                posflat[pl.ds(c * 16, 16)] = pos
                return tuple(new_runs)
            return lax.fori_loop(0, 16, chunk, run0)

        run0 = tuple(ebase[e] for e in range(E))

        for j in range(2):
            for cc in range(8):
                tokb[j, pl.ds(cc * 16, 16)] = (base + j * 128 + cc * 16) + lane

        def emit(pos_out_h, p_h):
            for j in range(2):
                for cc in range(8):
                    posb2d[j, pl.ds(cc * 16, 16)] = posflat[
                        pl.ds(j * 128 + cc * 16, 16)]
            pltpu.sync_copy(posb2d, pos_out_h.at[pl.ds(sid * 2, 2)])
            pltpu.sync_copy(p_h.at[pl.ds(base, 256)], pflat)
            for j in range(2):
                for cc in range(8):
                    pb2d[j, pl.ds(cc * 16, 16)] = pflat[
                        pl.ds(j * 128 + cc * 16, 16)]
            for j in range(2):
                pltpu.async_copy(tokb.at[j], tos_h.at[posb2d.at[j]],
                                 sem).wait()
                pltpu.async_copy(pb2d.at[j], psc_h.at[posb2d.at[j]],
                                 sem).wait()

        run1 = scan(e0v, run0)
        emit(pos0_h, p0_h)
        scan(e1v, run1)
        emit(pos1_h, p1_h)


def _routing(idx0, idx1, p0, p1, hist):
    mesh = plsc.VectorSubcoreMesh(core_axis_name="c", subcore_axis_name="s")
    f = pl.kernel(
        _routing_body,
        out_type=[
            jax.ShapeDtypeStruct((N // 128, 128), _i32),
            jax.ShapeDtypeStruct((N // 128, 128), _i32),
            jax.ShapeDtypeStruct((S,), _i32),
            jax.ShapeDtypeStruct((S,), _f32),
        ],
        mesh=mesh,
        scratch_types=[
            pltpu.VMEM((256,), _i32),
            pltpu.VMEM((256,), _i32),
            pltpu.VMEM((256,), _f32),
            pltpu.VMEM((256,), _i32),
            pltpu.VMEM((2, 128), _i32),
            pltpu.VMEM((2, 128), _i32),
            pltpu.VMEM((2, 128), _f32),
            pltpu.VMEM((NSC * 16,), _i32),
            pltpu.VMEM((S // NSC,), _i32),
            pltpu.SemaphoreType.DMA,
        ],
        compiler_params=pltpu.CompilerParams(needs_layout_passes=False),
    )
    return f(idx0, idx1, p0, p1, hist)


_GCH = 32
_GR = (S // NW) // _GCH


def _gather_body(xf_h, tos_h, xs_h, idx2, rows0, rows1,
                 sg0, sg1, sc0, sc1):
    cid = lax.axis_index("c")
    sid = lax.axis_index("s")
    wid = sid * NCORE + cid
    base = wid * (S // NW)
    pltpu.sync_copy(tos_h.at[pl.ds(base, S // NW)], idx2)

    rows = (rows0, rows1)
    sg = (sg0, sg1)
    sc = (sc0, sc1)
    ga = {}
    cp = {}

    def copy_out(r):
        b = r & 1
        ga[r].wait()
        cp[r] = pltpu.async_copy(
            rows[b], xs_h.at[pl.ds(base + r * _GCH, _GCH), :], sc[b])

    for r in range(_GR):
        b = r & 1
        if r >= 2:
            cp[r - 2].wait()
        ga[r] = pltpu.async_copy(
            xf_h.at[idx2.at[pl.ds(r * _GCH, _GCH)]], rows[b], sg[b])
        if r >= 1:
            copy_out(r - 1)
    copy_out(_GR - 1)
    cp[_GR - 2].wait()
    cp[_GR - 1].wait()


def _gather(xf, tos2):
    mesh = plsc.VectorSubcoreMesh(core_axis_name="c", subcore_axis_name="s")
    f = pl.kernel(
        _gather_body,
        out_type=[jax.ShapeDtypeStruct((S, D), _f32)],
        mesh=mesh,
        scratch_types=[
            pltpu.VMEM((S // NW,), _i32),
            pltpu.VMEM((_GCH, D), _f32),
            pltpu.VMEM((_GCH, D), _f32),
            pltpu.SemaphoreType.DMA,
            pltpu.SemaphoreType.DMA,
            pltpu.SemaphoreType.DMA,
            pltpu.SemaphoreType.DMA,
        ],
        compiler_params=pltpu.CompilerParams(needs_layout_passes=False),
    )
    return f(xf, tos2)[0]


NH = 4
HB = H // NH


def _ffn_body(te_ref, act_ref, xs_ref, psc_ref,
              w1_ref, w2_ref, wp_ref, b1_ref, b2_ref, bp_ref, ysin_ref,
              ys_ref):
    h = pl.program_id(0)
    t = pl.program_id(1)

    @pl.when(act_ref[t] == 1)
    def _():
        xb = xs_ref[...].astype(jnp.bfloat16)
        h1 = jnp.dot(xb, w1_ref[0].astype(jnp.bfloat16),
                     preferred_element_type=_f32) + b1_ref[0]
        h2 = jnp.dot(xb, w2_ref[0].astype(jnp.bfloat16),
                     preferred_element_type=_f32) + b2_ref[0]
        g = (h1 * jax.nn.sigmoid(h1)) * h2
        part = jnp.dot(g.astype(jnp.bfloat16), wp_ref[0].astype(jnp.bfloat16),
                       preferred_element_type=_f32)
        acc = ysin_ref[...] + part
        final = (acc + bp_ref[0]) * psc_ref[...]
        ys_ref[...] = jnp.where(h == NH - 1, final, acc)


def _ffn(te, act, xs, psc, w1b, w2b, wpb, b1, b2, bp, ysin):
    grid_spec = pltpu.PrefetchScalarGridSpec(
        num_scalar_prefetch=2,
        grid=(NH, PT),
        in_specs=[
            pl.BlockSpec((TM, D), lambda h, t, te, a: (t, 0)),
            pl.BlockSpec((TM, 1), lambda h, t, te, a: (t, 0)),
            pl.BlockSpec((1, D, HB), lambda h, t, te, a: (te[t], 0, h)),
            pl.BlockSpec((1, D, HB), lambda h, t, te, a: (te[t], 0, h)),
            pl.BlockSpec((1, HB, D), lambda h, t, te, a: (te[t], h, 0)),
            pl.BlockSpec((1, 1, HB), lambda h, t, te, a: (te[t], 0, h)),
            pl.BlockSpec((1, 1, HB), lambda h, t, te, a: (te[t], 0, h)),
            pl.BlockSpec((1, 1, D), lambda h, t, te, a: (te[t], 0, 0)),
            pl.BlockSpec((TM, D), lambda h, t, te, a: (t, 0)),
        ],
        out_specs=pl.BlockSpec((TM, D), lambda h, t, te, a: (t, 0)),
    )
    return pl.pallas_call(
        _ffn_body,
        grid_spec=grid_spec,
        out_shape=jax.ShapeDtypeStruct((S, D), _f32),
        input_output_aliases={10: 0},
    )(te, act, xs, psc, w1b, w2b, wpb, b1, b2, bp, ysin)


_CR = 8
_CCH = (N // NW) // _CR


def _combine_body(ys_h, pos0_h, pos1_h, a0_h, a1_h, i0v, i1v,
                  a00, a01, a10, a11, sa0, sa1, sb0, sb1, so0, so1,
                  sp0, sp1):
    cid = lax.axis_index("c")
    sid = lax.axis_index("s")
    wid = sid * NCORE + cid
    pltpu.sync_copy(pos0_h.at[pl.ds(wid, 1)], i0v)
    pltpu.sync_copy(pos1_h.at[pl.ds(wid, 1)], i1v)

    a0 = (a00, a01)
    a1 = (a10, a11)
    sa = (sa0, sa1)
    sb = (sb0, sb1)
    so = (so0, so1)
    sp = (sp0, sp1)
    g0 = {}
    g1 = {}
    cp0 = {}
    cp1 = {}

    def copy_out(r):
        b = r & 1
        g0[r].wait()
        g1[r].wait()
        dst = pl.ds(wid * 128 + r * _CCH, _CCH)
        cp0[r] = pltpu.async_copy(a0[b], a0_h.at[dst, :], so[b])
        cp1[r] = pltpu.async_copy(a1[b], a1_h.at[dst, :], sp[b])

    for r in range(_CR):
        b = r & 1
        if r >= 2:
            cp0[r - 2].wait()
            cp1[r - 2].wait()
        g0[r] = pltpu.async_copy(
            ys_h.at[i0v.at[0, pl.ds(r * _CCH, _CCH)]], a0[b], sa[b])
        g1[r] = pltpu.async_copy(
            ys_h.at[i1v.at[0, pl.ds(r * _CCH, _CCH)]], a1[b], sb[b])
        if r >= 1:
            copy_out(r - 1)
    copy_out(_CR - 1)
    for r in (_CR - 2, _CR - 1):
        cp0[r].wait()
        cp1[r].wait()


def _combine(ys, pos0, pos1):
    mesh = plsc.VectorSubcoreMesh(core_axis_name="c", subcore_axis_name="s")
    f = pl.kernel(
        _combine_body,
        out_type=[
            jax.ShapeDtypeStruct((N, D), _f32),
            jax.ShapeDtypeStruct((N, D), _f32),
        ],
        mesh=mesh,
        scratch_types=[
            pltpu.VMEM((1, 128), _i32),
            pltpu.VMEM((1, 128), _i32),
            pltpu.VMEM((_CCH, D), _f32),
            pltpu.VMEM((_CCH, D), _f32),
            pltpu.VMEM((_CCH, D), _f32),
            pltpu.VMEM((_CCH, D), _f32),
            pltpu.SemaphoreType.DMA,
            pltpu.SemaphoreType.DMA,
            pltpu.SemaphoreType.DMA,
            pltpu.SemaphoreType.DMA,
            pltpu.SemaphoreType.DMA,
            pltpu.SemaphoreType.DMA,
            pltpu.SemaphoreType.DMA,
            pltpu.SemaphoreType.DMA,
        ],
        compiler_params=pltpu.CompilerParams(needs_layout_passes=False),
    )
    return f(ys, pos0, pos1)


def _add_body(a_ref, b_ref, o_ref):
    o_ref[...] = a_ref[...] + b_ref[...]


def _add(a, b):
    return pl.pallas_call(
        _add_body,
        grid=(N // GT,),
        in_specs=[pl.BlockSpec((GT, D), lambda t: (t, 0)),
                  pl.BlockSpec((GT, D), lambda t: (t, 0))],
        out_specs=pl.BlockSpec((GT, D), lambda t: (t, 0)),
        out_shape=jax.ShapeDtypeStruct((N, D), _f32),
    )(a, b)


def kernel(x, gate_w, gate_b, w1, b1, w2, b2, wp, bp):
    xf = x.reshape(N, D)
    (idx0, idx1, p0, p1, te128, act128, aux, hist) = _gate(xf, gate_w,
                                                           gate_b)
    pos0, pos1, tos, psc = _routing(idx0.reshape(N), idx1.reshape(N),
                                    p0.reshape(N), p1.reshape(N),
                                    hist.reshape(NSC * 16))
    xs = _gather(xf, tos)
    ys = _ffn(te128[0, :PT], act128[0, :PT], xs, psc.reshape(S, 1),
              w1, w2, wp, b1.reshape(E, 1, H),
              b2.reshape(E, 1, H), bp.reshape(E, 1, D),
              jnp.zeros((S, D), _f32))
    a0, a1 = _combine(ys, pos0, pos1)
    out = _add(a0, a1)
    return out.reshape(B, T, D), aux[0, 0]
